# baseline (speedup 1.0000x reference)
import functools

import jax
import jax.numpy as jnp
from jax import lax
from jax.experimental import pallas as pl
from jax.experimental.pallas import tpu as pltpu
from jax.experimental.pallas import tpu_sc as plsc

NC, NS, L = 2, 16, 16
NW = NC * NS
B = 16384
D = 64
OUT_D = 2 * D + 1
BPW = B // NW
NCHUNK = BPW // L
NBUCKETS = 10
GATHER_CHUNK = 128
NG = BPW // GATHER_CHUNK

_mesh = plsc.VectorSubcoreMesh(core_axis_name="c", subcore_axis_name="s")


@functools.partial(
    pl.kernel,
    out_type=jax.ShapeDtypeStruct((B, OUT_D), jnp.float32),
    mesh=_mesh,
    compiler_params=pltpu.CompilerParams(
        use_tc_tiling_on_sc=False, needs_layout_passes=False),
    scratch_types=[
        pltpu.VMEM((BPW,), jnp.int32),
        pltpu.VMEM((BPW,), jnp.float32),
        pltpu.VMEM((BPW,), jnp.int32),
        pltpu.VMEM((BPW,), jnp.int32),
        pltpu.VMEM((BPW, 1), jnp.float32),
        pltpu.VMEM((NBUCKETS * L,), jnp.float32),
        pltpu.VMEM((2 * L,), jnp.float32),
        pltpu.VMEM((BPW, D), jnp.float32),
        pltpu.VMEM((BPW, D), jnp.float32),
        pltpu.SemaphoreType.DMA,
        pltpu.SemaphoreType.DMA,
    ],
)
def _user_model_sc(uid_hbm, age_hbm, utab_hbm, atab_hbm, bb_hbm, stats_hbm,
                   out_hbm, uid_v, age_v, idx_v, bins_v, norm_v, bb_v,
                   stats_v, urows_v, arows_v, usem, asem):
    wid = lax.axis_index("s") * NC + lax.axis_index("c")
    base = wid * BPW

    pltpu.sync_copy(uid_hbm.at[pl.ds(base, BPW)], uid_v)
    pltpu.sync_copy(age_hbm.at[pl.ds(base, BPW)], age_v)
    pltpu.sync_copy(bb_hbm, bb_v)
    pltpu.sync_copy(stats_hbm, stats_v)

    mean = stats_v[pl.ds(0, L)]
    inv_std = stats_v[pl.ds(L, L)]
    lane = lax.iota(jnp.int32, L)
    zero = jnp.zeros((L,), jnp.int32)
    for c in range(NCHUNK):
        sl = pl.ds(c * L, L)
        idx_v[sl] = uid_v[sl] + 1
        a = age_v[sl]
        cnt = jnp.zeros((L,), jnp.int32)
        ones = jnp.ones((L,), jnp.int32)
        zeros32 = jnp.zeros((L,), jnp.int32)
        for i in range(NBUCKETS):
            cnt = cnt + jnp.where(a >= bb_v[pl.ds(i * L, L)], ones, zeros32)
        bins_v[sl] = cnt
        row_idx = lane + jnp.full((L,), c * L, jnp.int32)
        plsc.store_scatter(norm_v, [row_idx, zero], (a - mean) * inv_std)

    ucopies = []
    acopies = []
    for g in range(NG):
        gs = pl.ds(g * GATHER_CHUNK, GATHER_CHUNK)
        ucopies.append(
            pltpu.async_copy(utab_hbm.at[idx_v.at[gs]], urows_v.at[gs], usem))
        acopies.append(
            pltpu.async_copy(atab_hbm.at[bins_v.at[gs]], arows_v.at[gs], asem))
    for cp in ucopies:
        cp.wait()
    for cp in acopies:
        cp.wait()

    rows = pl.ds(base, BPW)
    pltpu.sync_copy(urows_v, out_hbm.at[rows, pl.ds(0, D)])
    pltpu.sync_copy(arows_v, out_hbm.at[rows, pl.ds(D, D)])
    pltpu.sync_copy(norm_v, out_hbm.at[rows, pl.ds(2 * D, 1)])


def kernel(user_id, age, user_table, age_table, age_buckets, age_mean, age_var):
    bb = jnp.broadcast_to(age_buckets[:, None], (NBUCKETS, L)).reshape(-1)
    inv_std = lax.rsqrt(age_var.astype(jnp.float32))
    stats = jnp.concatenate([
        jnp.broadcast_to(jnp.asarray(age_mean, jnp.float32), (L,)),
        jnp.broadcast_to(inv_std, (L,)),
    ])
    return _user_model_sc(user_id, age, user_table, age_table, bb, stats)
